# baseline (speedup 1.0000x reference)
import functools

import jax
import jax.numpy as jnp
from jax import lax
from jax.experimental import pallas as pl
from jax.experimental.pallas import tpu as pltpu
from jax.experimental.pallas import tpu_sc as plsc

VOCAB = 1000000
HID = 64
B = 4096
L = 200
C0 = 128
C1 = L - C0

NC = 2
NS = 16
NW = NC * NS
BPW = B // NW
NVR = HID // 16
NCHUNK = 2 * BPW

BT = 32768
BH = BT // 2
TGRID = (VOCAB + BT - 1) // BT
VROWS = TGRID * BT


def _transpose_body(in_ref, out_ref):
    x = in_ref[...]
    z = jnp.concatenate([x[:, :BH], x[:, BH:]], axis=0)
    out_ref[...] = z.T


def _linearize(table_t):
    return pl.pallas_call(
        _transpose_body,
        grid=(TGRID,),
        in_specs=[pl.BlockSpec((HID, BT), lambda i: (0, i))],
        out_specs=pl.BlockSpec((BH, 128), lambda i: (i, 0)),
        out_shape=jax.ShapeDtypeStruct((TGRID * BH, 128), jnp.float32),
    )(table_t)


NBUF = 12


def _pool_body(inp_hbm, table_hbm, out_hbm, idx_v, *rest):
    rbufs = rest[:NBUF]
    bow_v = rest[NBUF]
    sems = rest[NBUF + 1:]
    wid = lax.axis_index("s") * NC + lax.axis_index("c")
    base = wid * BPW
    pltpu.sync_copy(inp_hbm.at[pl.ds(base, BPW)], idx_v)
    sizes = tuple(C0 if b % 2 == 0 else C1 for b in range(NBUF))
    offs = tuple(0 if b % 2 == 0 else C0 for b in range(NBUF))

    def start(b, chunk):
        bag = chunk // 2
        return pltpu.async_copy(
            table_hbm.at[idx_v.at[bag, pl.ds(offs[b], sizes[b])]],
            rbufs[b], sems[b])

    def wait(b):
        pltpu.make_async_copy(
            table_hbm.at[idx_v.at[0, pl.ds(offs[b], sizes[b])]],
            rbufs[b], sems[b]).wait()

    def acc_chunk(rows_ref, n, accs):
        def row(i, accs):
            return tuple(a + rows_ref[i, pl.ds(16 * c, 16)]
                         for c, a in enumerate(accs))
        return lax.fori_loop(0, n, row, accs, unroll=8)

    for b in range(NBUF):
        start(b, b)

    scale = jnp.float32(1.0 / L)

    def bags(j, carry):
        c0 = NBUF * j
        for h in range(NBUF // 2):
            accs = tuple(jnp.zeros((16,), jnp.float32) for _ in range(NVR))
            wait(2 * h)
            accs = acc_chunk(rbufs[2 * h], C0, accs)
            start(2 * h, jnp.minimum(c0 + NBUF + 2 * h, NCHUNK - 2))
            wait(2 * h + 1)
            accs = acc_chunk(rbufs[2 * h + 1], C1, accs)
            bag = (NBUF // 2) * j + h
            for c in range(NVR):
                bow_v[bag, pl.ds(16 * c, 16)] = accs[c] * scale
            start(2 * h + 1, jnp.minimum(c0 + NBUF + 2 * h + 1, NCHUNK - 1))
        return carry

    lax.fori_loop(0, BPW // (NBUF // 2), bags, 0)
    for b in range(NBUF):
        wait(b)
    pltpu.sync_copy(bow_v, out_hbm.at[pl.ds(base, BPW)])


_pool = functools.partial(
    pl.kernel,
    mesh=plsc.VectorSubcoreMesh(core_axis_name="c", subcore_axis_name="s"),
    out_type=jax.ShapeDtypeStruct((B, HID), jnp.float32),
    compiler_params=pltpu.CompilerParams(use_tc_tiling_on_sc=False),
    scratch_types=(
        [pltpu.VMEM((BPW, L), jnp.int32)]
        + [pltpu.VMEM((C0 if b % 2 == 0 else C1, HID), jnp.float32)
           for b in range(NBUF)]
        + [pltpu.VMEM((BPW, HID), jnp.float32)]
        + [pltpu.SemaphoreType.DMA] * NBUF
    ),
)(_pool_body)


def _head_body(bow_ref, labels_ref, w1_ref, b1_ref, gamma_ref, beta_ref,
               w2_ref, b2_ref, logits_ref, loss_ref):
    bow = bow_ref[...]
    h = jnp.dot(bow, w1_ref[...], preferred_element_type=jnp.float32)
    h = h + b1_ref[...]
    mean = jnp.mean(h, axis=0, keepdims=True)
    d = h - mean
    var = jnp.mean(d * d, axis=0, keepdims=True)
    hn = d * lax.rsqrt(var + 1e-5) * gamma_ref[...] + beta_ref[...]
    x = jnp.maximum(hn, 0.0)
    logits = jnp.dot(x, w2_ref[...], preferred_element_type=jnp.float32)
    logits = logits + b2_ref[...]
    logits_ref[...] = logits
    lab = labels_ref[...]
    per = (jnp.maximum(logits, 0.0) - logits * lab
           + jnp.log1p(jnp.exp(-jnp.abs(logits))))
    loss_ref[...] = (jnp.sum(per) * (1.0 / B))[None, None]


def kernel(inp, labels, table, W1, b1, gamma, beta, W2, b2):
    inp32 = inp.astype(jnp.int32)
    idx = ((inp32 & ~jnp.int32(BT - 1))
           | ((inp32 & (BH - 1)) << 1)
           | ((inp32 >> (BH.bit_length() - 1)) & 1))
    table_lin = _linearize(table.T).reshape(VROWS, HID)
    bow = _pool(idx, table_lin)
    logits2, loss2 = pl.pallas_call(
        _head_body,
        out_shape=(
            jax.ShapeDtypeStruct((B, 1), jnp.float32),
            jax.ShapeDtypeStruct((1, 1), jnp.float32),
        ),
    )(bow, labels.reshape(B, 1), W1, b1.reshape(1, HID),
      gamma.reshape(1, HID), beta.reshape(1, HID), W2, b2.reshape(1, 1))
    return (loss2[0, 0], logits2[:, 0])


# speedup vs baseline: 1.0396x; 1.0396x over previous
import functools

import jax
import jax.numpy as jnp
from jax import lax
from jax.experimental import pallas as pl
from jax.experimental.pallas import tpu as pltpu
from jax.experimental.pallas import tpu_sc as plsc

VOCAB = 1000000
HID = 64
B = 4096
L = 200
C0 = 128
C1 = L - C0

NC = 2
NS = 16
NW = NC * NS
BPW = B // NW
NVR = HID // 16
NCHUNK = 2 * BPW

BT = 32768
BH = BT // 2
TGRID = (VOCAB + BT - 1) // BT
VROWS = TGRID * BT


def _transpose_body(in_ref, out_ref):
    x = in_ref[...]
    z = jnp.concatenate([x[:, :BH], x[:, BH:]], axis=0)
    out_ref[...] = z.T


def _linearize(table_t):
    return pl.pallas_call(
        _transpose_body,
        grid=(TGRID,),
        in_specs=[pl.BlockSpec((HID, BT), lambda i: (0, i))],
        out_specs=pl.BlockSpec((BH, 128), lambda i: (i, 0)),
        out_shape=jax.ShapeDtypeStruct((TGRID * BH, 128), jnp.float32),
    )(table_t)


NBUF = 8


def _pool_body(inp_hbm, table_hbm, out_hbm, idx_v,
               rb0, rb1, rb2, rb3, rb4, rb5, rb6, rb7, bow_v,
               sem0, sem1, sem2, sem3, sem4, sem5, sem6, sem7):
    wid = lax.axis_index("s") * NC + lax.axis_index("c")
    base = wid * BPW
    pltpu.sync_copy(inp_hbm.at[pl.ds(base, BPW)], idx_v)

    rbufs = (rb0, rb1, rb2, rb3, rb4, rb5, rb6, rb7)
    sems = (sem0, sem1, sem2, sem3, sem4, sem5, sem6, sem7)
    sizes = tuple(C0 if b % 2 == 0 else C1 for b in range(NBUF))
    offs = tuple(0 if b % 2 == 0 else C0 for b in range(NBUF))

    def start(b, chunk):
        bag = chunk // 2
        return pltpu.async_copy(
            table_hbm.at[idx_v.at[bag, pl.ds(offs[b], sizes[b])]],
            rbufs[b], sems[b])

    def wait(b):
        pltpu.make_async_copy(
            table_hbm.at[idx_v.at[0, pl.ds(offs[b], sizes[b])]],
            rbufs[b], sems[b]).wait()

    def acc_chunk(rows_ref, n, accs):
        def row(i, accs):
            return tuple(a + rows_ref[i, pl.ds(16 * c, 16)]
                         for c, a in enumerate(accs))
        return lax.fori_loop(0, n, row, accs, unroll=8)

    for b in range(NBUF):
        start(b, b)

    scale = jnp.float32(1.0 / L)

    def bags(j, carry):
        c0 = NBUF * j
        for h in range(NBUF // 2):
            accs = tuple(jnp.zeros((16,), jnp.float32) for _ in range(NVR))
            wait(2 * h)
            accs = acc_chunk(rbufs[2 * h], C0, accs)
            start(2 * h, jnp.minimum(c0 + NBUF + 2 * h, NCHUNK - 2))
            wait(2 * h + 1)
            accs = acc_chunk(rbufs[2 * h + 1], C1, accs)
            bag = (NBUF // 2) * j + h
            for c in range(NVR):
                bow_v[bag, pl.ds(16 * c, 16)] = accs[c] * scale
            start(2 * h + 1, jnp.minimum(c0 + NBUF + 2 * h + 1, NCHUNK - 1))
        return carry

    lax.fori_loop(0, BPW // (NBUF // 2), bags, 0)
    for b in range(NBUF):
        wait(b)
    pltpu.sync_copy(bow_v, out_hbm.at[pl.ds(base, BPW)])


_pool = functools.partial(
    pl.kernel,
    mesh=plsc.VectorSubcoreMesh(core_axis_name="c", subcore_axis_name="s"),
    out_type=jax.ShapeDtypeStruct((B, HID), jnp.float32),
    compiler_params=pltpu.CompilerParams(use_tc_tiling_on_sc=False),
    scratch_types=(
        [pltpu.VMEM((BPW, L), jnp.int32)]
        + [pltpu.VMEM((C0 if b % 2 == 0 else C1, HID), jnp.float32)
           for b in range(NBUF)]
        + [pltpu.VMEM((BPW, HID), jnp.float32)]
        + [pltpu.SemaphoreType.DMA] * NBUF
    ),
)(_pool_body)


def _head_body(bow_ref, labels_ref, w1_ref, b1_ref, gamma_ref, beta_ref,
               w2_ref, b2_ref, logits_ref, loss_ref):
    bow = bow_ref[...]
    h = jnp.dot(bow, w1_ref[...], preferred_element_type=jnp.float32)
    h = h + b1_ref[...]
    mean = jnp.mean(h, axis=0, keepdims=True)
    d = h - mean
    var = jnp.mean(d * d, axis=0, keepdims=True)
    hn = d * lax.rsqrt(var + 1e-5) * gamma_ref[...] + beta_ref[...]
    x = jnp.maximum(hn, 0.0)
    logits = jnp.dot(x, w2_ref[...], preferred_element_type=jnp.float32)
    logits = logits + b2_ref[...]
    logits_ref[...] = logits
    lab = labels_ref[...]
    per = (jnp.maximum(logits, 0.0) - logits * lab
           + jnp.log1p(jnp.exp(-jnp.abs(logits))))
    loss_ref[...] = (jnp.sum(per) * (1.0 / B))[None, None]


def kernel(inp, labels, table, W1, b1, gamma, beta, W2, b2):
    inp32 = inp.astype(jnp.int32)
    idx = ((inp32 & ~jnp.int32(BT - 1))
           | ((inp32 & (BH - 1)) << 1)
           | ((inp32 >> (BH.bit_length() - 1)) & 1))
    table_lin = _linearize(table.T).reshape(VROWS, HID)
    bow = _pool(idx, table_lin)
    logits2, loss2 = pl.pallas_call(
        _head_body,
        out_shape=(
            jax.ShapeDtypeStruct((B, 1), jnp.float32),
            jax.ShapeDtypeStruct((1, 1), jnp.float32),
        ),
    )(bow, labels.reshape(B, 1), W1, b1.reshape(1, HID),
      gamma.reshape(1, HID), beta.reshape(1, HID), W2, b2.reshape(1, 1))
    return (loss2[0, 0], logits2[:, 0])
